# baseline (speedup 1.0000x reference)
import functools

import jax
import jax.numpy as jnp
from jax import lax
from jax.experimental import pallas as pl
from jax.experimental.pallas import tpu as pltpu
from jax.experimental.pallas import tpu_sc as plsc


def _relayout(mt, V, D, NC, NS):
    NW = NC * NS
    W = 512
    S_full = V // W
    REM = V - S_full * W
    per_w = S_full // NW
    extra = S_full - per_w * NW

    mesh = plsc.VectorSubcoreMesh(core_axis_name="c", subcore_axis_name="s")

    @functools.partial(
        pl.kernel,
        mesh=mesh,
        out_type=jax.ShapeDtypeStruct((V * D,), jnp.float32),
        scratch_types=[
            pltpu.VMEM((D, W), jnp.float32),
            pltpu.VMEM((D, W), jnp.float32),
            pltpu.VMEM((W * D,), jnp.float32),
            pltpu.VMEM((W * D,), jnp.float32),
            pltpu.VMEM((D, REM), jnp.float32),
            pltpu.VMEM((REM * D,), jnp.float32),
            pltpu.SemaphoreType.DMA,
            pltpu.SemaphoreType.DMA,
            pltpu.SemaphoreType.DMA,
            pltpu.SemaphoreType.DMA,
        ],
        compiler_params=pltpu.CompilerParams(needs_layout_passes=False),
    )
    def relayout_kernel(mt_hbm, out_hbm, in_v0, in_v1, out_v0, out_v1,
                        tin_v, tout_v, g0, g1, w0, w1):
        in_v = (in_v0, in_v1)
        out_v = (out_v0, out_v1)
        wid = lax.axis_index("s") * NC + lax.axis_index("c")
        gsem = (g0, g1)
        wsem = (w0, w1)
        lanes = lax.iota(jnp.int32, 16)

        def transpose_slab(src, dst, width):
            dvec1 = 16 + lanes

            @plsc.parallel_loop(0, width, step=1, unroll=8)
            def body(c):
                fc = jnp.full((16,), c, jnp.int32)
                x0 = plsc.load_gather(src, [lanes, fc])
                x1 = plsc.load_gather(src, [dvec1, fc])
                dst[pl.ds(c * D, 16)] = x0
                dst[pl.ds(c * D + 16, 16)] = x1

        def slab_start(s, slot):
            return pltpu.async_copy(
                mt_hbm.at[:, pl.ds(s * W, W)], in_v[slot], gsem[slot])

        def slab_write(s, slot):
            return pltpu.async_copy(
                out_v[slot], out_hbm.at[pl.ds(s * W * D, W * D)],
                wsem[slot])

        base = wid * per_w

        hg = {0: slab_start(base, 0)}
        hw = {}
        for i in range(per_w):
            slot = i % 2
            if i + 1 < per_w:
                nslot = 1 - slot
                if i >= 1:
                    hw[i - 1].wait()
                hg[i + 1] = slab_start(base + i + 1, nslot)
            hg[i].wait()
            transpose_slab(in_v[slot], out_v[slot], W)
            hw[i] = slab_write(base + i, slot)
        if per_w >= 2:
            hw[per_w - 2].wait()
        hw[per_w - 1].wait()

        @pl.when(wid < extra)
        def _():
            s = NW * per_w + wid
            pltpu.async_copy(
                mt_hbm.at[:, pl.ds(s * W, W)], in_v[0], g0).wait()
            transpose_slab(in_v[0], out_v[0], W)
            pltpu.async_copy(
                out_v[0], out_hbm.at[pl.ds(s * W * D, W * D)], w0).wait()

        @pl.when(wid == NW - 1)
        def _():
            c0 = S_full * W
            pltpu.async_copy(
                mt_hbm.at[:, pl.ds(c0, REM)], tin_v, g1).wait()
            transpose_slab(tin_v, tout_v, REM)
            pltpu.async_copy(
                tout_v, out_hbm.at[pl.ds(c0 * D, REM * D)], w1).wait()

    return relayout_kernel(mt)


def _gather_tiled_out(idx_flat, table, B0, B1, V, D, NC, NS):
    NW = NC * NS
    B = B0 * B1
    b_per_w = B // NW
    t_per_w = B0 // NW
    TB = B0 // 128
    DB = D // 8
    TQ = 32
    CH = TQ * B1
    n_ch = t_per_w // TQ

    mesh = plsc.VectorSubcoreMesh(core_axis_name="c", subcore_axis_name="s")

    @functools.partial(
        pl.kernel,
        mesh=mesh,
        out_type=jax.ShapeDtypeStruct((B1, DB, TB, 8, 128), jnp.float32),
        scratch_types=[
            pltpu.VMEM((b_per_w,), jnp.int32),
            pltpu.VMEM((2, CH, D), jnp.float32),
            pltpu.VMEM((2 * DB, B1, 8, TQ), jnp.float32),
            pltpu.SemaphoreType.DMA,
            pltpu.SemaphoreType.DMA,
            pltpu.SemaphoreType.DMA,
            pltpu.SemaphoreType.DMA,
        ],
        compiler_params=pltpu.CompilerParams(use_tc_tiling_on_sc=False, needs_layout_passes=False),
    )
    def gather_kernel(idx_hbm, table_hbm, out_hbm, idx_v, rows_v, stage_v,
                      g0, g1, w0, w1):
        wid = lax.axis_index("s") * NC + lax.axis_index("c")
        base = wid * b_per_w
        gsem = (g0, g1)
        wsem = (w0, w1)
        lanes = lax.iota(jnp.int32, 16)
        lanes_b1 = lanes * B1

        pltpu.sync_copy(idx_hbm.at[pl.ds(base, b_per_w)], idx_v)

        def gather(j, slot):
            return pltpu.async_copy(
                table_hbm.at[idx_v.at[pl.ds(j * CH, CH)]], rows_v.at[slot],
                gsem[slot])

        def transpose_chunk(slot):
            src = rows_v.at[slot]

            @plsc.parallel_loop(0, B1 * D * 2, step=1, unroll=8)
            def body(k):
                tg = k & 1
                pd = k >> 1
                d = pd & (D - 1)
                p = pd >> 5
                db = d >> 3
                ds = d & 7
                row = lanes_b1 + (tg * (16 * B1) + p)
                x = plsc.load_gather(src, [row, jnp.full((16,), d, jnp.int32)])
                plsc.store_scatter(
                    stage_v,
                    [jnp.full((16,), slot * DB + db, jnp.int32),
                     jnp.full((16,), p, jnp.int32),
                     jnp.full((16,), ds, jnp.int32),
                     lanes + tg * 16],
                    x)

        def writeout(j, slot):
            tb = wid * (t_per_w // 128) + j // 4
            tl0 = (j % 4) * TQ
            handles = []
            for db in range(DB):
                handles.append(pltpu.async_copy(
                    stage_v.at[slot * DB + db],
                    out_hbm.at[:, db, tb, :, pl.ds(tl0, TQ)],
                    wsem[slot]))
            return handles

        hg = {0: gather(0, 0)}
        hw = {}
        for j in range(n_ch):
            slot = j % 2
            if j + 1 < n_ch:
                hg[j + 1] = gather(j + 1, 1 - slot)
            hg[j].wait()
            if j >= 2:
                for h in hw[j - 2]:
                    h.wait()
            transpose_chunk(slot)
            hw[j] = writeout(j, slot)
        for h in hw[n_ch - 2]:
            h.wait()
        for h in hw[n_ch - 1]:
            h.wait()

    return gather_kernel(idx_flat, table)


def kernel(token_ids, matrix):
    B0, B1 = token_ids.shape
    V, D = matrix.shape
    B = B0 * B1
    info = plsc.get_sparse_core_info()
    NC, NS = info.num_cores, info.num_subcores

    relay = _relayout(matrix.T, V, D, NC, NS)
    y = _gather_tiled_out(token_ids.reshape(B), relay.reshape(V, D),
                          B0, B1, V, D, NC, NS)
    yr = jnp.transpose(y, (2, 4, 0, 1, 3))
    return yr.reshape(B0, B1, D)


# speedup vs baseline: 1.0637x; 1.0637x over previous
import functools

import jax
import jax.numpy as jnp
from jax import lax
from jax.experimental import pallas as pl
from jax.experimental.pallas import tpu as pltpu
from jax.experimental.pallas import tpu_sc as plsc


def _relayout(mt, V, D, NC, NS):
    NW = NC * NS
    W = 512
    S_full = V // W
    REM = V - S_full * W
    per_w = S_full // NW
    extra = S_full - per_w * NW

    mesh = plsc.VectorSubcoreMesh(core_axis_name="c", subcore_axis_name="s")

    @functools.partial(
        pl.kernel,
        mesh=mesh,
        out_type=jax.ShapeDtypeStruct((V * D,), jnp.float32),
        scratch_types=[
            pltpu.VMEM((D, W), jnp.float32),
            pltpu.VMEM((D, W), jnp.float32),
            pltpu.VMEM((W * D,), jnp.float32),
            pltpu.VMEM((W * D,), jnp.float32),
            pltpu.VMEM((D, REM), jnp.float32),
            pltpu.VMEM((REM * D,), jnp.float32),
            pltpu.SemaphoreType.DMA,
            pltpu.SemaphoreType.DMA,
            pltpu.SemaphoreType.DMA,
            pltpu.SemaphoreType.DMA,
        ],
        compiler_params=pltpu.CompilerParams(needs_layout_passes=False),
    )
    def relayout_kernel(mt_hbm, out_hbm, in_v0, in_v1, out_v0, out_v1,
                        tin_v, tout_v, g0, g1, w0, w1):
        in_v = (in_v0, in_v1)
        out_v = (out_v0, out_v1)
        wid = lax.axis_index("s") * NC + lax.axis_index("c")
        gsem = (g0, g1)
        wsem = (w0, w1)
        lanes = lax.iota(jnp.int32, 16)

        def transpose_slab(src, dst, width):
            for g in range(D // 16):
                dvec = g * 16 + lanes

                @plsc.parallel_loop(0, width, step=1, unroll=8)
                def body(c):
                    x = plsc.load_gather(
                        src, [dvec, jnp.full((16,), c, jnp.int32)])
                    dst[pl.ds(c * D + g * 16, 16)] = x

        def slab_start(s, slot):
            return pltpu.async_copy(
                mt_hbm.at[:, pl.ds(s * W, W)], in_v[slot], gsem[slot])

        def slab_write(s, slot):
            return pltpu.async_copy(
                out_v[slot], out_hbm.at[pl.ds(s * W * D, W * D)],
                wsem[slot])

        base = wid * per_w

        hg = {0: slab_start(base, 0)}
        hw = {}
        for i in range(per_w):
            slot = i % 2
            if i + 1 < per_w:
                nslot = 1 - slot
                if i >= 1:
                    hw[i - 1].wait()
                hg[i + 1] = slab_start(base + i + 1, nslot)
            hg[i].wait()
            transpose_slab(in_v[slot], out_v[slot], W)
            hw[i] = slab_write(base + i, slot)
        if per_w >= 2:
            hw[per_w - 2].wait()
        hw[per_w - 1].wait()

        @pl.when(wid < extra)
        def _():
            s = NW * per_w + wid
            pltpu.async_copy(
                mt_hbm.at[:, pl.ds(s * W, W)], in_v[0], g0).wait()
            transpose_slab(in_v[0], out_v[0], W)
            pltpu.async_copy(
                out_v[0], out_hbm.at[pl.ds(s * W * D, W * D)], w0).wait()

        @pl.when(wid == NW - 1)
        def _():
            c0 = S_full * W
            pltpu.async_copy(
                mt_hbm.at[:, pl.ds(c0, REM)], tin_v, g1).wait()
            transpose_slab(tin_v, tout_v, REM)
            pltpu.async_copy(
                tout_v, out_hbm.at[pl.ds(c0 * D, REM * D)], w1).wait()

    return relayout_kernel(mt)


def _gather_tiled_out(idx_flat, table, B0, B1, V, D, NC, NS):
    NW = NC * NS
    B = B0 * B1
    b_per_w = B // NW
    t_per_w = B0 // NW
    TB = B0 // 128
    DB = D // 8
    TQ = 32
    CH = TQ * B1
    n_ch = t_per_w // TQ

    mesh = plsc.VectorSubcoreMesh(core_axis_name="c", subcore_axis_name="s")

    @functools.partial(
        pl.kernel,
        mesh=mesh,
        out_type=jax.ShapeDtypeStruct((B1, DB, TB, 8, 128), jnp.float32),
        scratch_types=[
            pltpu.VMEM((b_per_w,), jnp.int32),
            pltpu.VMEM((2, CH, D), jnp.float32),
            pltpu.VMEM((2 * DB, B1, 8, TQ), jnp.float32),
            pltpu.SemaphoreType.DMA,
            pltpu.SemaphoreType.DMA,
            pltpu.SemaphoreType.DMA,
            pltpu.SemaphoreType.DMA,
        ],
        compiler_params=pltpu.CompilerParams(use_tc_tiling_on_sc=False, needs_layout_passes=False),
    )
    def gather_kernel(idx_hbm, table_hbm, out_hbm, idx_v, rows_v, stage_v,
                      g0, g1, w0, w1):
        wid = lax.axis_index("s") * NC + lax.axis_index("c")
        base = wid * b_per_w
        gsem = (g0, g1)
        wsem = (w0, w1)
        lanes = lax.iota(jnp.int32, 16)
        lanes_b1 = lanes * B1

        pltpu.sync_copy(idx_hbm.at[pl.ds(base, b_per_w)], idx_v)

        def gather(j, slot):
            return pltpu.async_copy(
                table_hbm.at[idx_v.at[pl.ds(j * CH, CH)]], rows_v.at[slot],
                gsem[slot])

        def transpose_chunk(slot):
            src = rows_v.at[slot]
            @plsc.parallel_loop(0, B1 * D * 2, step=1, unroll=8)
            def body(k):
                tg = k & 1
                pd = k >> 1
                d = pd & (D - 1)
                p = pd >> 5
                ds = d & 7
                row = lanes_b1 + (tg * (16 * B1) + p)
                x = plsc.load_gather(src, [row, jnp.full((16,), d, jnp.int32)])
                plsc.store_scatter(
                    stage_v,
                    [jnp.full((16,), slot * DB + (d >> 3), jnp.int32),
                     jnp.full((16,), p, jnp.int32),
                     jnp.full((16,), ds, jnp.int32),
                     lanes + tg * 16],
                    x)

        def writeout(j, slot):
            tb = wid * (t_per_w // 128) + j // 4
            tl0 = (j % 4) * TQ
            handles = []
            for db in range(DB):
                handles.append(pltpu.async_copy(
                    stage_v.at[slot * DB + db],
                    out_hbm.at[:, db, tb, :, pl.ds(tl0, TQ)],
                    wsem[slot]))
            return handles

        hg = {0: gather(0, 0)}
        hw = {}
        for j in range(n_ch):
            slot = j % 2
            if j + 1 < n_ch:
                hg[j + 1] = gather(j + 1, 1 - slot)
            hg[j].wait()
            if j >= 2:
                for h in hw[j - 2]:
                    h.wait()
            transpose_chunk(slot)
            hw[j] = writeout(j, slot)
        for h in hw[n_ch - 2]:
            h.wait()
        for h in hw[n_ch - 1]:
            h.wait()

    return gather_kernel(idx_flat, table)


def kernel(token_ids, matrix):
    B0, B1 = token_ids.shape
    V, D = matrix.shape
    B = B0 * B1
    info = plsc.get_sparse_core_info()
    NC, NS = info.num_cores, info.num_subcores

    relay = _relayout(matrix.T, V, D, NC, NS)
    y = _gather_tiled_out(token_ids.reshape(B), relay.reshape(V, D),
                          B0, B1, V, D, NC, NS)
    yr = jnp.transpose(y, (2, 4, 0, 1, 3))
    return yr.reshape(B0, B1, D)
